# baseline (speedup 1.0000x reference)
import functools

import jax
import jax.numpy as jnp
from jax import lax
from jax.experimental import pallas as pl
from jax.experimental.pallas import tpu as pltpu
from jax.experimental.pallas import tpu_sc as plsc

B = 1024
L = 50
D = 64
V = 119547

NC = 2
NS = 16
NW = NC * NS
BPW = B // NW

LANES = 16


def _sc_embed_mean(x, embed_table):
    mesh = plsc.VectorSubcoreMesh(core_axis_name="c", subcore_axis_name="s")

    @functools.partial(
        pl.kernel,
        out_type=jax.ShapeDtypeStruct((B, D), jnp.float32),
        mesh=mesh,
        compiler_params=pltpu.CompilerParams(use_tc_tiling_on_sc=False),
        scratch_types=[
            pltpu.VMEM((BPW, L), jnp.int32),
            pltpu.VMEM((L, D), jnp.float32),
            pltpu.VMEM((BPW, D), jnp.float32),
            pltpu.SemaphoreType.DMA,
        ],
    )
    def k(x_hbm, table_hbm, out_hbm, idx_v, rows_v, h_v, sem):
        wid = lax.axis_index("s") * NC + lax.axis_index("c")
        base = wid * BPW
        pltpu.sync_copy(x_hbm.at[pl.ds(base, BPW)], idx_v)

        @pl.loop(0, BPW)
        def _(r):
            pltpu.async_copy(table_hbm.at[idx_v.at[r]], rows_v, sem).wait()
            for c in range(0, D, LANES):
                acc = rows_v[0, pl.ds(c, LANES)]
                for l in range(1, L):
                    acc = acc + rows_v[l, pl.ds(c, LANES)]
                h_v[r, pl.ds(c, LANES)] = acc * (1.0 / L)

        pltpu.sync_copy(h_v, out_hbm.at[pl.ds(base, BPW)])

    return k(x, embed_table)


VT = 2048
GRID_V = (V + VT - 1) // VT


def _tc_logits(h, W, b2):
    def body(h_ref, w_ref, b_ref, o_ref):
        o_ref[...] = lax.dot_general(
            h_ref[...], w_ref[...],
            (((1,), (1,)), ((), ())),
            preferred_element_type=jnp.float32,
        ) + b_ref[...]

    return pl.pallas_call(
        body,
        grid=(GRID_V,),
        in_specs=[
            pl.BlockSpec((B, D), lambda i: (0, 0)),
            pl.BlockSpec((VT, D), lambda i: (i, 0)),
            pl.BlockSpec((1, VT), lambda i: (0, i)),
        ],
        out_specs=pl.BlockSpec((B, VT), lambda i: (0, i)),
        out_shape=jax.ShapeDtypeStruct((B, V), jnp.float32),
    )(h, W, b2)


def kernel(x, embed_table, W, b):
    h = _sc_embed_mean(x, embed_table)
    return _tc_logits(h, W, b.reshape(1, V))


# speedup vs baseline: 1.0029x; 1.0029x over previous
import functools

import jax
import jax.numpy as jnp
from jax import lax
from jax.experimental import pallas as pl
from jax.experimental.pallas import tpu as pltpu
from jax.experimental.pallas import tpu_sc as plsc

B = 1024
L = 50
D = 64
V = 119547

NC = 2
NS = 16
NW = NC * NS
BPW = B // NW

LANES = 16


def _sc_embed_mean(x, embed_table):
    mesh = plsc.VectorSubcoreMesh(core_axis_name="c", subcore_axis_name="s")

    @functools.partial(
        pl.kernel,
        out_type=jax.ShapeDtypeStruct((B, D), jnp.float32),
        mesh=mesh,
        compiler_params=pltpu.CompilerParams(use_tc_tiling_on_sc=False),
        scratch_types=[
            pltpu.VMEM((BPW, L), jnp.int32),
            pltpu.VMEM((L, D), jnp.float32),
            pltpu.VMEM((BPW, D), jnp.float32),
            pltpu.SemaphoreType.DMA,
        ],
    )
    def k(x_hbm, table_hbm, out_hbm, idx_v, rows_v, h_v, sem):
        wid = lax.axis_index("s") * NC + lax.axis_index("c")
        base = wid * BPW
        pltpu.sync_copy(x_hbm.at[pl.ds(base, BPW)], idx_v)

        @pl.loop(0, BPW)
        def _(r):
            pltpu.async_copy(table_hbm.at[idx_v.at[r]], rows_v, sem).wait()
            for c in range(0, D, LANES):
                acc = rows_v[0, pl.ds(c, LANES)]
                for l in range(1, L):
                    acc = acc + rows_v[l, pl.ds(c, LANES)]
                h_v[r, pl.ds(c, LANES)] = acc * (1.0 / L)

        pltpu.sync_copy(h_v, out_hbm.at[pl.ds(base, BPW)])

    return k(x, embed_table)


VT = 4096
GRID_V = (V + VT - 1) // VT


def _tc_logits(h, W, b2):
    def body(h_ref, w_ref, b_ref, o_ref):
        o_ref[...] = lax.dot_general(
            h_ref[...], w_ref[...],
            (((1,), (1,)), ((), ())),
            preferred_element_type=jnp.float32,
        ) + b_ref[...]

    return pl.pallas_call(
        body,
        grid=(GRID_V,),
        in_specs=[
            pl.BlockSpec((B, D), lambda i: (0, 0)),
            pl.BlockSpec((VT, D), lambda i: (i, 0)),
            pl.BlockSpec((1, VT), lambda i: (0, i)),
        ],
        out_specs=pl.BlockSpec((B, VT), lambda i: (0, i)),
        out_shape=jax.ShapeDtypeStruct((B, V), jnp.float32),
    )(h, W, b2)


def kernel(x, embed_table, W, b):
    h = _sc_embed_mean(x, embed_table)
    return _tc_logits(h, W, b.reshape(1, V))
